# baseline (speedup 1.0000x reference)
import functools

import jax
import jax.numpy as jnp
from jax import lax
from jax.experimental import pallas as pl
from jax.experimental.pallas import tpu as pltpu

N = 10000
E = 160000
H = 128
NRBF = 128
L = 2
T = 2
RC = 5.0
NG = 16


def _out_stage_kernel(x_ref, w1_ref, b1_ref, w2_ref, b2_ref, batch_ref, out_ref):
    b = pl.program_id(0)
    x = x_ref[...]
    h = jax.nn.silu(x @ w1_ref[...] + b1_ref[...]) * (1.0 / 0.6)
    pa = h @ w2_ref[...] + b2_ref[...]
    bt = batch_ref[0, 0, :]
    oh = (bt[:, None] == lax.broadcasted_iota(jnp.int32, (1, NG), 1)).astype(jnp.float32)
    partial = lax.dot_general(pa, oh, (((0,), (0,)), ((), ())))

    @pl.when(b == 0)
    def _():
        out_ref[...] = partial

    @pl.when(b != 0)
    def _():
        out_ref[...] = out_ref[...] + partial


def _out_stage(x, Wo1, bo1, Wo2, bo2, batch):
    B = 1000
    nb = N // B
    batch3 = batch.astype(jnp.int32).reshape(nb, 1, B)
    out = pl.pallas_call(
        _out_stage_kernel,
        grid=(nb,),
        in_specs=[
            pl.BlockSpec((B, H), lambda b: (b, 0)),
            pl.BlockSpec((H, H // 2), lambda b: (0, 0)),
            pl.BlockSpec((1, H // 2), lambda b: (0, 0)),
            pl.BlockSpec((H // 2, 1), lambda b: (0, 0)),
            pl.BlockSpec((1, 1), lambda b: (0, 0)),
            pl.BlockSpec((1, 1, B), lambda b: (b, 0, 0)),
        ],
        out_specs=pl.BlockSpec((1, NG), lambda b: (0, 0)),
        out_shape=jax.ShapeDtypeStruct((1, NG), jnp.float32),
    )(x, Wo1, bo1.reshape(1, -1), Wo2, bo2.reshape(1, 1), batch3)
    return out.reshape(NG)


def _radial_basis(dist):
    d = dist / RC
    p = 5.0
    env = (1.0 - ((p + 1.0) * (p + 2.0) / 2.0) * d ** p
           + p * (p + 2.0) * d ** (p + 1.0)
           - (p * (p + 1.0) / 2.0) * d ** (p + 2.0))
    env = jnp.where(d < 1.0, env, 0.0)
    offsets = jnp.linspace(0.0, 1.0, NRBF)
    coeff = -0.5 / (offsets[1] - offsets[0]) ** 2
    rbf = jnp.exp(coeff * (d[:, None] - offsets[None, :]) ** 2)
    return rbf * env[:, None]


def kernel(pos, embed, Wm1, bm1, Wm2, bm2, Wf, bf, Wo1, bo1, Wo2, bo2,
           atomic_number, edge_index, batch):
    j = edge_index[0]
    i = edge_index[1]
    distance_vec = pos[j] - pos[i]
    edge_dist = jnp.linalg.norm(distance_vec, axis=-1)
    edge_dist = jnp.where(jnp.isclose(edge_dist, 0.0, atol=1e-6), 1e-6, edge_dist)
    edge_vec = distance_vec / edge_dist[:, None]
    edge_embed = _radial_basis(edge_dist)
    x = embed[atomic_number]
    vec = jnp.zeros((x.shape[0], 3, H), dtype=x.dtype)
    dst_type = atomic_number[i]
    for l in range(L):
        Phi = jnp.stack([
            jax.nn.silu(x @ Wm1[l, t] + bm1[l, t]) @ Wm2[l, t] + bm2[l, t]
            for t in range(T)
        ])
        phi_e = Phi[dst_type, j]
        filt0 = edge_embed @ Wf[l, 0] + bf[l, 0]
        filt1 = edge_embed @ Wf[l, 1] + bf[l, 1]
        filt = jnp.where((dst_type == 0)[:, None], filt0, filt1)
        m = phi_e * filt
        dx = m[:, :H]
        dv1 = m[:, H:2 * H]
        dv2 = m[:, 2 * H:]
        dvec_msg = dv1[:, None, :] * vec[j] + dv2[:, None, :] * edge_vec[:, :, None]
        x = jax.ops.segment_sum(dx, i, num_segments=N)
        vec = jax.ops.segment_sum(dvec_msg, i, num_segments=N)
    return _out_stage(x, Wo1, bo1, Wo2, bo2, batch)


# speedup vs baseline: 5.6832x; 5.6832x over previous
import functools

import jax
import jax.numpy as jnp
from jax import lax
from jax.experimental import pallas as pl
from jax.experimental.pallas import tpu as pltpu
from jax.experimental.pallas import tpu_sc as plsc

N = 10000
E = 160000
H = 128
NRBF = 128
L = 2
T = 2
RC = 5.0
NG = 16

NC = 2
NS = 16
NW = NC * NS
CH = 128
EPAD = 163840
EW = EPAD // NW
NCHUNK = EW // CH
NPAD = 10240
NSUB = NPAD // NS
TRASH = N
BE = 2048
BN = 1000


def _sc_mesh():
    return plsc.VectorSubcoreMesh(core_axis_name="c", subcore_axis_name="s",
                                  num_cores=NC, num_subcores=NS)


def _pre_body(posx_h, posy_h, posz_h, an_h, j_h, i_h, d2_h, idx_h, te_h,
              jv, iv, xj, yj, zj, xi, yi, zi, tv, d2v, idxv, tev, sem):
    w = lax.axis_index("s") * NC + lax.axis_index("c")
    base = w * EW
    pltpu.sync_copy(j_h.at[pl.ds(base, EW)], jv)
    pltpu.sync_copy(i_h.at[pl.ds(base, EW)], iv)

    def chunk(c, carry):
        s = c * CH
        jc = jv.at[pl.ds(s, CH)]
        ic = iv.at[pl.ds(s, CH)]
        ds = pl.ds(s, CH)
        descs = [
            pltpu.async_copy(posx_h.at[jc], xj.at[ds], sem),
            pltpu.async_copy(posy_h.at[jc], yj.at[ds], sem),
            pltpu.async_copy(posz_h.at[jc], zj.at[ds], sem),
            pltpu.async_copy(posx_h.at[ic], xi.at[ds], sem),
            pltpu.async_copy(posy_h.at[ic], yi.at[ds], sem),
            pltpu.async_copy(posz_h.at[ic], zi.at[ds], sem),
            pltpu.async_copy(an_h.at[ic], tv.at[ds], sem),
        ]
        for d in descs:
            d.wait()
        return carry

    lax.fori_loop(0, NCHUNK, chunk, 0)

    def q_body(q, carry):
        s = q * 16
        sl = pl.ds(s, 16)
        dx = xj[sl] - xi[sl]
        dy = yj[sl] - yi[sl]
        dz = zj[sl] - zi[sl]
        d2v[sl] = dx * dx + dy * dy + dz * dz
        t16 = tv[sl]
        idxv[sl] = t16 * N + jv[sl]
        tev[sl] = t16
        return carry

    lax.fori_loop(0, EW // 16, q_body, 0)
    pltpu.sync_copy(d2v, d2_h.at[pl.ds(base, EW)])
    pltpu.sync_copy(idxv, idx_h.at[pl.ds(base, EW)])
    pltpu.sync_copy(tev, te_h.at[pl.ds(base, EW)])


def _pre(posx, posy, posz, an, jp, ip):
    k = pl.kernel(
        _pre_body,
        out_type=[jax.ShapeDtypeStruct((EPAD,), jnp.float32),
                  jax.ShapeDtypeStruct((EPAD,), jnp.int32),
                  jax.ShapeDtypeStruct((EPAD,), jnp.int32)],
        mesh=_sc_mesh(),
        scratch_types=[pltpu.VMEM((EW,), jnp.int32),
                       pltpu.VMEM((EW,), jnp.int32),
                       pltpu.VMEM((EW,), jnp.float32),
                       pltpu.VMEM((EW,), jnp.float32),
                       pltpu.VMEM((EW,), jnp.float32),
                       pltpu.VMEM((EW,), jnp.float32),
                       pltpu.VMEM((EW,), jnp.float32),
                       pltpu.VMEM((EW,), jnp.float32),
                       pltpu.VMEM((EW,), jnp.int32),
                       pltpu.VMEM((EW,), jnp.float32),
                       pltpu.VMEM((EW,), jnp.int32),
                       pltpu.VMEM((EW,), jnp.int32),
                       pltpu.SemaphoreType.DMA],
    )
    return k(posx, posy, posz, an, jp, ip)


def _gather_body(tab_h, idx_h, out_h, idx_v, rows_v, sem):
    w = lax.axis_index("s") * NC + lax.axis_index("c")
    base = w * NCHUNK
    pltpu.sync_copy(idx_h.at[pl.ds(base, NCHUNK)], idx_v)

    def c_body(c, carry):
        pltpu.async_copy(tab_h.at[idx_v.at[c]], rows_v, sem).wait()
        pltpu.sync_copy(rows_v, out_h.at[pl.ds((base + c) * CH, CH)])
        return carry

    lax.fori_loop(0, NCHUNK, c_body, 0)


def _gather(tab, idx2d):
    k = pl.kernel(
        _gather_body,
        out_type=jax.ShapeDtypeStruct((EPAD, H), jnp.float32),
        mesh=_sc_mesh(),
        scratch_types=[pltpu.VMEM((NCHUNK, CH), jnp.int32),
                       pltpu.VMEM((CH, H), jnp.float32),
                       pltpu.SemaphoreType.DMA],
    )
    return k(tab, idx2d)


def _scatter_body(dx_h, idx_h, zeros_h, out_h, idx_v, rows_v, acc):
    cid = lax.axis_index("c")
    sid = lax.axis_index("s")
    w = sid * NC + cid
    base = w * NCHUNK
    r0 = sid * NSUB
    pltpu.sync_copy(zeros_h.at[pl.ds(r0, NSUB)], acc.at[pl.ds(r0, NSUB)])
    pltpu.sync_copy(idx_h.at[pl.ds(base, NCHUNK)], idx_v)
    plsc.subcore_barrier()

    def c_body(c, carry):
        pltpu.sync_copy(dx_h.at[pl.ds((base + c) * CH, CH)], rows_v)
        pltpu.sync_copy(rows_v, acc.at[idx_v.at[c]], add=True)
        return carry

    lax.fori_loop(0, NCHUNK, c_body, 0)
    plsc.subcore_barrier()
    pltpu.sync_copy(acc.at[pl.ds(r0, NSUB)], out_h.at[cid, pl.ds(r0, NSUB)])


def _scatter(dx, iscat, zerosN):
    k = pl.kernel(
        _scatter_body,
        out_type=jax.ShapeDtypeStruct((NC, NPAD, H), jnp.float32),
        mesh=_sc_mesh(),
        scratch_types=[pltpu.VMEM((NCHUNK, CH), jnp.int32),
                       pltpu.VMEM((CH, H), jnp.float32),
                       pltpu.VMEM_SHARED((NPAD, H), jnp.float32)],
    )
    return k(dx, iscat, zerosN)


def _x0_kernel(an_ref, emb_ref, out_ref):
    a = an_ref[0, 0, :]
    m = a[:, None] == 0
    x0 = jnp.where(m, emb_ref[0:1, :], emb_ref[1:2, :])
    out_ref[0, :, :] = x0
    out_ref[1, :, :] = jnp.zeros_like(x0)


def _x0(an3, embed):
    nb = N // BN
    return pl.pallas_call(
        _x0_kernel,
        grid=(nb,),
        in_specs=[pl.BlockSpec((1, 1, BN), lambda b: (b, 0, 0)),
                  pl.BlockSpec((T, H), lambda b: (0, 0))],
        out_specs=pl.BlockSpec((2, BN, H), lambda b: (0, b, 0)),
        out_shape=jax.ShapeDtypeStruct((2, NPAD, H), jnp.float32),
    )(an3, embed)


def _phi_kernel(xp_ref, w10, b10, w20, b20, w11, b11, w21, b21, out_ref):
    x = xp_ref[0, :, :] + xp_ref[1, :, :]
    h0 = jax.nn.silu(jnp.dot(x, w10[...], preferred_element_type=jnp.float32)
                     + b10[...])
    p0 = jnp.dot(h0, w20[...], preferred_element_type=jnp.float32) + b20[...]
    h1 = jax.nn.silu(jnp.dot(x, w11[...], preferred_element_type=jnp.float32)
                     + b11[...])
    p1 = jnp.dot(h1, w21[...], preferred_element_type=jnp.float32) + b21[...]
    out_ref[0, :, :] = p0
    out_ref[1, :, :] = p1


def _phi(xp, w10, b10, w20, b20, w11, b11, w21, b21):
    nb = N // BN
    wspec = pl.BlockSpec((H, H), lambda b: (0, 0))
    bspec = pl.BlockSpec((1, H), lambda b: (0, 0))
    return pl.pallas_call(
        _phi_kernel,
        grid=(nb,),
        in_specs=[pl.BlockSpec((2, BN, H), lambda b: (0, b, 0)),
                  wspec, bspec, wspec, bspec, wspec, bspec, wspec, bspec],
        out_specs=pl.BlockSpec((2, BN, H), lambda b: (0, b, 0)),
        out_shape=jax.ShapeDtypeStruct((2, N, H), jnp.float32),
    )(xp, w10, b10, w20, b20, w11, b11, w21, b21)


def _dx_kernel(phi_ref, d2_ref, te_ref, wf0, bf0, wf1, bf1, out_ref):
    d2 = d2_ref[0, 0, :]
    d = jnp.sqrt(d2)
    d = jnp.where(d <= 1e-6, 1e-6, d)
    u = d * (1.0 / RC)
    u2 = u * u
    u4 = u2 * u2
    u5 = u4 * u
    env = 1.0 - 21.0 * u5 + 35.0 * u5 * u - 15.0 * u5 * u2
    env = jnp.where(u < 1.0, env, 0.0)
    offs = lax.broadcasted_iota(jnp.int32, (1, NRBF), 1).astype(jnp.float32) * (
        1.0 / (NRBF - 1))
    delta = 1.0 / (NRBF - 1)
    coeff = -0.5 / (delta * delta)
    diff = u[:, None] - offs
    rbf = jnp.exp(coeff * (diff * diff)) * env[:, None]
    f0 = jnp.dot(rbf, wf0[...], preferred_element_type=jnp.float32) + bf0[...]
    f1 = jnp.dot(rbf, wf1[...], preferred_element_type=jnp.float32) + bf1[...]
    t = te_ref[0, 0, :]
    filt = jnp.where(t[:, None] == 0, f0, f1)
    out_ref[...] = phi_ref[...] * filt


def _dx(phi_e, d2r, ter, wf0, bf0, wf1, bf1):
    nbe = EPAD // BE
    wspec = pl.BlockSpec((NRBF, H), lambda b: (0, 0))
    bspec = pl.BlockSpec((1, H), lambda b: (0, 0))
    return pl.pallas_call(
        _dx_kernel,
        grid=(nbe,),
        in_specs=[pl.BlockSpec((BE, H), lambda b: (b, 0)),
                  pl.BlockSpec((1, 1, BE), lambda b: (b, 0, 0)),
                  pl.BlockSpec((1, 1, BE), lambda b: (b, 0, 0)),
                  wspec, bspec, wspec, bspec],
        out_specs=pl.BlockSpec((BE, H), lambda b: (b, 0)),
        out_shape=jax.ShapeDtypeStruct((EPAD, H), jnp.float32),
    )(phi_e, d2r, ter, wf0, bf0, wf1, bf1)


def _out_kernel(xp_ref, w1_ref, b1_ref, w2_ref, b2_ref, batch_ref, out_ref):
    b = pl.program_id(0)
    x = xp_ref[0, :, :] + xp_ref[1, :, :]
    h = jax.nn.silu(jnp.dot(x, w1_ref[...], preferred_element_type=jnp.float32)
                    + b1_ref[...]) * (1.0 / 0.6)
    pa = jnp.dot(h, w2_ref[...], preferred_element_type=jnp.float32) + b2_ref[...]
    bt = batch_ref[0, 0, :]
    oh = (bt[:, None] == lax.broadcasted_iota(jnp.int32, (1, NG), 1)
          ).astype(jnp.float32)
    partial = lax.dot_general(pa, oh, (((0,), (0,)), ((), ())))

    @pl.when(b == 0)
    def _():
        out_ref[...] = partial

    @pl.when(b != 0)
    def _():
        out_ref[...] = out_ref[...] + partial


def _out_stage(xp, Wo1, bo1, Wo2, bo2, batch3):
    nb = N // BN
    out = pl.pallas_call(
        _out_kernel,
        grid=(nb,),
        in_specs=[
            pl.BlockSpec((2, BN, H), lambda b: (0, b, 0)),
            pl.BlockSpec((H, H // 2), lambda b: (0, 0)),
            pl.BlockSpec((1, H // 2), lambda b: (0, 0)),
            pl.BlockSpec((H // 2, 1), lambda b: (0, 0)),
            pl.BlockSpec((1, 1), lambda b: (0, 0)),
            pl.BlockSpec((1, 1, BN), lambda b: (b, 0, 0)),
        ],
        out_specs=pl.BlockSpec((1, NG), lambda b: (0, 0)),
        out_shape=jax.ShapeDtypeStruct((1, NG), jnp.float32),
    )(xp, Wo1, bo1.reshape(1, -1), Wo2, bo2.reshape(1, 1), batch3)
    return out.reshape(NG)


def kernel(pos, embed, Wm1, bm1, Wm2, bm2, Wf, bf, Wo1, bo1, Wo2, bo2,
           atomic_number, edge_index, batch):
    f32 = jnp.float32
    pos = pos.astype(f32)
    posx = pos[:, 0]
    posy = pos[:, 1]
    posz = pos[:, 2]
    an = atomic_number.astype(jnp.int32)
    j = edge_index[0].astype(jnp.int32)
    i = edge_index[1].astype(jnp.int32)
    pad = EPAD - E
    jp = jnp.concatenate([j, jnp.zeros((pad,), jnp.int32)])
    ip_g = jnp.concatenate([i, jnp.zeros((pad,), jnp.int32)])
    ip_s = jnp.concatenate([i, jnp.full((pad,), TRASH, jnp.int32)])

    d2, idx_phi, te = _pre(posx, posy, posz, an, jp, ip_g)

    an3 = an.reshape(N // BN, 1, BN)
    xp = _x0(an3, embed.astype(f32))

    d2r = d2.reshape(EPAD // BE, 1, BE)
    ter = te.reshape(EPAD // BE, 1, BE)
    idx2d = idx_phi.reshape(EPAD // CH, CH)
    iscat = ip_s.reshape(EPAD // CH, CH)
    zerosN = jnp.zeros((NPAD, H), f32)
    batch3 = batch.astype(jnp.int32).reshape(N // BN, 1, BN)

    for l in range(L):
        phitab = _phi(xp,
                      Wm1[l, 0], bm1[l, 0].reshape(1, H),
                      Wm2[l, 0, :, :H], bm2[l, 0, :H].reshape(1, H),
                      Wm1[l, 1], bm1[l, 1].reshape(1, H),
                      Wm2[l, 1, :, :H], bm2[l, 1, :H].reshape(1, H))
        phi_e = _gather(phitab.reshape(2 * N, H), idx2d)
        dxv = _dx(phi_e, d2r, ter,
                  Wf[l, 0, :, :H], bf[l, 0, :H].reshape(1, H),
                  Wf[l, 1, :, :H], bf[l, 1, :H].reshape(1, H))
        xp = _scatter(dxv, iscat, zerosN)

    return _out_stage(xp, Wo1, bo1, Wo2, bo2, batch3)


# speedup vs baseline: 6.4794x; 1.1401x over previous
import functools

import jax
import jax.numpy as jnp
from jax import lax
from jax.experimental import pallas as pl
from jax.experimental.pallas import tpu as pltpu
from jax.experimental.pallas import tpu_sc as plsc

N = 10000
E = 160000
H = 128
NRBF = 128
L = 2
T = 2
RC = 5.0
NG = 16

NC = 2
NS = 16
NW = NC * NS
CH = 128
EPAD = 163840
EW = EPAD // NW
NCHUNK = EW // CH
NPAD = 10240
NSUB = NPAD // NS
TRASH = N
BE = 2048
BN = 1000


def _sc_mesh():
    return plsc.VectorSubcoreMesh(core_axis_name="c", subcore_axis_name="s",
                                  num_cores=NC, num_subcores=NS)


def _pre_body(posx_h, posy_h, posz_h, an_h, j_h, i_h, d2_h, idx_h, te_h,
              jv, iv, xj, yj, zj, xi, yi, zi, tv, d2v, idxv, tev, sem):
    w = lax.axis_index("s") * NC + lax.axis_index("c")
    base = w * EW
    pltpu.sync_copy(j_h.at[pl.ds(base, EW)], jv)
    pltpu.sync_copy(i_h.at[pl.ds(base, EW)], iv)

    def chunk(c, carry):
        s = c * CH
        jc = jv.at[pl.ds(s, CH)]
        ic = iv.at[pl.ds(s, CH)]
        ds = pl.ds(s, CH)
        descs = [
            pltpu.async_copy(posx_h.at[jc], xj.at[ds], sem),
            pltpu.async_copy(posy_h.at[jc], yj.at[ds], sem),
            pltpu.async_copy(posz_h.at[jc], zj.at[ds], sem),
            pltpu.async_copy(posx_h.at[ic], xi.at[ds], sem),
            pltpu.async_copy(posy_h.at[ic], yi.at[ds], sem),
            pltpu.async_copy(posz_h.at[ic], zi.at[ds], sem),
            pltpu.async_copy(an_h.at[ic], tv.at[ds], sem),
        ]
        for d in descs:
            d.wait()
        return carry

    lax.fori_loop(0, NCHUNK, chunk, 0)

    def q_body(q, carry):
        s = q * 16
        sl = pl.ds(s, 16)
        dx = xj[sl] - xi[sl]
        dy = yj[sl] - yi[sl]
        dz = zj[sl] - zi[sl]
        d2v[sl] = dx * dx + dy * dy + dz * dz
        t16 = tv[sl]
        idxv[sl] = t16 * N + jv[sl]
        tev[sl] = t16
        return carry

    lax.fori_loop(0, EW // 16, q_body, 0)
    pltpu.sync_copy(d2v, d2_h.at[pl.ds(base, EW)])
    pltpu.sync_copy(idxv, idx_h.at[pl.ds(base, EW)])
    pltpu.sync_copy(tev, te_h.at[pl.ds(base, EW)])


def _pre(posx, posy, posz, an, jp, ip):
    k = pl.kernel(
        _pre_body,
        out_type=[jax.ShapeDtypeStruct((EPAD,), jnp.float32),
                  jax.ShapeDtypeStruct((EPAD,), jnp.int32),
                  jax.ShapeDtypeStruct((EPAD,), jnp.int32)],
        mesh=_sc_mesh(),
        scratch_types=[pltpu.VMEM((EW,), jnp.int32),
                       pltpu.VMEM((EW,), jnp.int32),
                       pltpu.VMEM((EW,), jnp.float32),
                       pltpu.VMEM((EW,), jnp.float32),
                       pltpu.VMEM((EW,), jnp.float32),
                       pltpu.VMEM((EW,), jnp.float32),
                       pltpu.VMEM((EW,), jnp.float32),
                       pltpu.VMEM((EW,), jnp.float32),
                       pltpu.VMEM((EW,), jnp.int32),
                       pltpu.VMEM((EW,), jnp.float32),
                       pltpu.VMEM((EW,), jnp.int32),
                       pltpu.VMEM((EW,), jnp.int32),
                       pltpu.SemaphoreType.DMA],
    )
    return k(posx, posy, posz, an, jp, ip)


def _gather_body(tab_h, idx_h, out_h, idx_v, rows_a, rows_b, sem_a, sem_b):
    w = lax.axis_index("s") * NC + lax.axis_index("c")
    base = w * NCHUNK
    pltpu.sync_copy(idx_h.at[pl.ds(base, NCHUNK)], idx_v)
    pltpu.async_copy(tab_h.at[idx_v.at[0]], rows_a, sem_a)

    def k_body(k, carry):
        c = 2 * k
        db = pltpu.async_copy(tab_h.at[idx_v.at[c + 1]], rows_b, sem_b)
        pltpu.make_async_copy(tab_h.at[idx_v.at[0]], rows_a, sem_a).wait()
        pltpu.sync_copy(rows_a, out_h.at[pl.ds((base + c) * CH, CH)])

        @pl.when(c + 2 < NCHUNK)
        def _():
            pltpu.async_copy(tab_h.at[idx_v.at[c + 2]], rows_a, sem_a)

        db.wait()
        pltpu.sync_copy(rows_b, out_h.at[pl.ds((base + c + 1) * CH, CH)])
        return carry

    lax.fori_loop(0, NCHUNK // 2, k_body, 0)


def _gather(tab, idx2d):
    k = pl.kernel(
        _gather_body,
        out_type=jax.ShapeDtypeStruct((EPAD, H), jnp.float32),
        mesh=_sc_mesh(),
        scratch_types=[pltpu.VMEM((NCHUNK, CH), jnp.int32),
                       pltpu.VMEM((CH, H), jnp.float32),
                       pltpu.VMEM((CH, H), jnp.float32),
                       pltpu.SemaphoreType.DMA,
                       pltpu.SemaphoreType.DMA],
    )
    return k(tab, idx2d)


def _scatter_body(dx_h, idx_h, zeros_h, out_h, idx_v, rows_a, rows_b, acc,
                  sem_a, sem_b):
    cid = lax.axis_index("c")
    sid = lax.axis_index("s")
    w = sid * NC + cid
    base = w * NCHUNK
    r0 = sid * NSUB
    pltpu.sync_copy(zeros_h.at[pl.ds(r0, NSUB)], acc.at[pl.ds(r0, NSUB)])
    pltpu.sync_copy(idx_h.at[pl.ds(base, NCHUNK)], idx_v)
    plsc.subcore_barrier()
    pltpu.async_copy(dx_h.at[pl.ds(base * CH, CH)], rows_a, sem_a)

    def k_body(k, carry):
        c = 2 * k
        db = pltpu.async_copy(dx_h.at[pl.ds((base + c + 1) * CH, CH)], rows_b,
                              sem_b)
        pltpu.make_async_copy(dx_h.at[pl.ds(base * CH, CH)], rows_a,
                              sem_a).wait()
        pltpu.sync_copy(rows_a, acc.at[idx_v.at[c]], add=True)

        @pl.when(c + 2 < NCHUNK)
        def _():
            pltpu.async_copy(dx_h.at[pl.ds((base + c + 2) * CH, CH)], rows_a,
                             sem_a)

        db.wait()
        pltpu.sync_copy(rows_b, acc.at[idx_v.at[c + 1]], add=True)
        return carry

    lax.fori_loop(0, NCHUNK // 2, k_body, 0)
    plsc.subcore_barrier()
    pltpu.sync_copy(acc.at[pl.ds(r0, NSUB)], out_h.at[cid, pl.ds(r0, NSUB)])


def _scatter(dx, iscat, zerosN):
    k = pl.kernel(
        _scatter_body,
        out_type=jax.ShapeDtypeStruct((NC, NPAD, H), jnp.float32),
        mesh=_sc_mesh(),
        scratch_types=[pltpu.VMEM((NCHUNK, CH), jnp.int32),
                       pltpu.VMEM((CH, H), jnp.float32),
                       pltpu.VMEM((CH, H), jnp.float32),
                       pltpu.VMEM_SHARED((NPAD, H), jnp.float32),
                       pltpu.SemaphoreType.DMA,
                       pltpu.SemaphoreType.DMA],
    )
    return k(dx, iscat, zerosN)


def _x0_kernel(an_ref, emb_ref, out_ref):
    a = an_ref[0, 0, :]
    m = a[:, None] == 0
    x0 = jnp.where(m, emb_ref[0:1, :], emb_ref[1:2, :])
    out_ref[0, :, :] = x0
    out_ref[1, :, :] = jnp.zeros_like(x0)


def _x0(an3, embed):
    nb = N // BN
    return pl.pallas_call(
        _x0_kernel,
        grid=(nb,),
        in_specs=[pl.BlockSpec((1, 1, BN), lambda b: (b, 0, 0)),
                  pl.BlockSpec((T, H), lambda b: (0, 0))],
        out_specs=pl.BlockSpec((2, BN, H), lambda b: (0, b, 0)),
        out_shape=jax.ShapeDtypeStruct((2, NPAD, H), jnp.float32),
    )(an3, embed)


def _phi_kernel(xp_ref, w10, b10, w20, b20, w11, b11, w21, b21, out_ref):
    x = xp_ref[0, :, :] + xp_ref[1, :, :]
    h0 = jax.nn.silu(jnp.dot(x, w10[...], preferred_element_type=jnp.float32)
                     + b10[...])
    p0 = jnp.dot(h0, w20[...], preferred_element_type=jnp.float32) + b20[...]
    h1 = jax.nn.silu(jnp.dot(x, w11[...], preferred_element_type=jnp.float32)
                     + b11[...])
    p1 = jnp.dot(h1, w21[...], preferred_element_type=jnp.float32) + b21[...]
    out_ref[0, :, :] = p0
    out_ref[1, :, :] = p1


def _phi(xp, w10, b10, w20, b20, w11, b11, w21, b21):
    nb = N // BN
    wspec = pl.BlockSpec((H, H), lambda b: (0, 0))
    bspec = pl.BlockSpec((1, H), lambda b: (0, 0))
    return pl.pallas_call(
        _phi_kernel,
        grid=(nb,),
        in_specs=[pl.BlockSpec((2, BN, H), lambda b: (0, b, 0)),
                  wspec, bspec, wspec, bspec, wspec, bspec, wspec, bspec],
        out_specs=pl.BlockSpec((2, BN, H), lambda b: (0, b, 0)),
        out_shape=jax.ShapeDtypeStruct((2, N, H), jnp.float32),
    )(xp, w10, b10, w20, b20, w11, b11, w21, b21)


def _dx_kernel(phi_ref, d2_ref, te_ref, wf0, bf0, wf1, bf1, out_ref):
    d2 = d2_ref[0, 0, :]
    d = jnp.sqrt(d2)
    d = jnp.where(d <= 1e-6, 1e-6, d)
    u = d * (1.0 / RC)
    u2 = u * u
    u4 = u2 * u2
    u5 = u4 * u
    env = 1.0 - 21.0 * u5 + 35.0 * u5 * u - 15.0 * u5 * u2
    env = jnp.where(u < 1.0, env, 0.0)
    offs = lax.broadcasted_iota(jnp.int32, (1, NRBF), 1).astype(jnp.float32) * (
        1.0 / (NRBF - 1))
    delta = 1.0 / (NRBF - 1)
    coeff = -0.5 / (delta * delta)
    diff = u[:, None] - offs
    rbf = jnp.exp(coeff * (diff * diff)) * env[:, None]
    f0 = jnp.dot(rbf, wf0[...], preferred_element_type=jnp.float32) + bf0[...]
    f1 = jnp.dot(rbf, wf1[...], preferred_element_type=jnp.float32) + bf1[...]
    t = te_ref[0, 0, :]
    filt = jnp.where(t[:, None] == 0, f0, f1)
    out_ref[...] = phi_ref[...] * filt


def _dx(phi_e, d2r, ter, wf0, bf0, wf1, bf1):
    nbe = EPAD // BE
    wspec = pl.BlockSpec((NRBF, H), lambda b: (0, 0))
    bspec = pl.BlockSpec((1, H), lambda b: (0, 0))
    return pl.pallas_call(
        _dx_kernel,
        grid=(nbe,),
        in_specs=[pl.BlockSpec((BE, H), lambda b: (b, 0)),
                  pl.BlockSpec((1, 1, BE), lambda b: (b, 0, 0)),
                  pl.BlockSpec((1, 1, BE), lambda b: (b, 0, 0)),
                  wspec, bspec, wspec, bspec],
        out_specs=pl.BlockSpec((BE, H), lambda b: (b, 0)),
        out_shape=jax.ShapeDtypeStruct((EPAD, H), jnp.float32),
    )(phi_e, d2r, ter, wf0, bf0, wf1, bf1)


def _out_kernel(xp_ref, w1_ref, b1_ref, w2_ref, b2_ref, batch_ref, out_ref):
    b = pl.program_id(0)
    x = xp_ref[0, :, :] + xp_ref[1, :, :]
    h = jax.nn.silu(jnp.dot(x, w1_ref[...], preferred_element_type=jnp.float32)
                    + b1_ref[...]) * (1.0 / 0.6)
    pa = jnp.dot(h, w2_ref[...], preferred_element_type=jnp.float32) + b2_ref[...]
    bt = batch_ref[0, 0, :]
    oh = (bt[:, None] == lax.broadcasted_iota(jnp.int32, (1, NG), 1)
          ).astype(jnp.float32)
    partial = lax.dot_general(pa, oh, (((0,), (0,)), ((), ())))

    @pl.when(b == 0)
    def _():
        out_ref[...] = partial

    @pl.when(b != 0)
    def _():
        out_ref[...] = out_ref[...] + partial


def _out_stage(xp, Wo1, bo1, Wo2, bo2, batch3):
    nb = N // BN
    out = pl.pallas_call(
        _out_kernel,
        grid=(nb,),
        in_specs=[
            pl.BlockSpec((2, BN, H), lambda b: (0, b, 0)),
            pl.BlockSpec((H, H // 2), lambda b: (0, 0)),
            pl.BlockSpec((1, H // 2), lambda b: (0, 0)),
            pl.BlockSpec((H // 2, 1), lambda b: (0, 0)),
            pl.BlockSpec((1, 1), lambda b: (0, 0)),
            pl.BlockSpec((1, 1, BN), lambda b: (b, 0, 0)),
        ],
        out_specs=pl.BlockSpec((1, NG), lambda b: (0, 0)),
        out_shape=jax.ShapeDtypeStruct((1, NG), jnp.float32),
    )(xp, Wo1, bo1.reshape(1, -1), Wo2, bo2.reshape(1, 1), batch3)
    return out.reshape(NG)


def kernel(pos, embed, Wm1, bm1, Wm2, bm2, Wf, bf, Wo1, bo1, Wo2, bo2,
           atomic_number, edge_index, batch):
    f32 = jnp.float32
    pos = pos.astype(f32)
    posx = pos[:, 0]
    posy = pos[:, 1]
    posz = pos[:, 2]
    an = atomic_number.astype(jnp.int32)
    j = edge_index[0].astype(jnp.int32)
    i = edge_index[1].astype(jnp.int32)
    pad = EPAD - E
    jp = jnp.concatenate([j, jnp.zeros((pad,), jnp.int32)])
    ip_g = jnp.concatenate([i, jnp.zeros((pad,), jnp.int32)])
    ip_s = jnp.concatenate([i, jnp.full((pad,), TRASH, jnp.int32)])

    d2, idx_phi, te = _pre(posx, posy, posz, an, jp, ip_g)

    an3 = an.reshape(N // BN, 1, BN)
    xp = _x0(an3, embed.astype(f32))

    d2r = d2.reshape(EPAD // BE, 1, BE)
    ter = te.reshape(EPAD // BE, 1, BE)
    idx2d = idx_phi.reshape(EPAD // CH, CH)
    iscat = ip_s.reshape(EPAD // CH, CH)
    zerosN = jnp.zeros((NPAD, H), f32)
    batch3 = batch.astype(jnp.int32).reshape(N // BN, 1, BN)

    for l in range(L):
        phitab = _phi(xp,
                      Wm1[l, 0], bm1[l, 0].reshape(1, H),
                      Wm2[l, 0, :, :H], bm2[l, 0, :H].reshape(1, H),
                      Wm1[l, 1], bm1[l, 1].reshape(1, H),
                      Wm2[l, 1, :, :H], bm2[l, 1, :H].reshape(1, H))
        phi_e = _gather(phitab.reshape(2 * N, H), idx2d)
        dxv = _dx(phi_e, d2r, ter,
                  Wf[l, 0, :, :H], bf[l, 0, :H].reshape(1, H),
                  Wf[l, 1, :, :H], bf[l, 1, :H].reshape(1, H))
        xp = _scatter(dxv, iscat, zerosN)

    return _out_stage(xp, Wo1, bo1, Wo2, bo2, batch3)


# speedup vs baseline: 9.0778x; 1.4010x over previous
import functools

import jax
import jax.numpy as jnp
from jax import lax
from jax.experimental import pallas as pl
from jax.experimental.pallas import tpu as pltpu
from jax.experimental.pallas import tpu_sc as plsc

N = 10000
E = 160000
H = 128
NRBF = 128
L = 2
T = 2
RC = 5.0
NG = 16

NC = 2
NS = 16
NW = NC * NS
CH = 64
EPAD = 163840
EW = EPAD // NW
NCHUNK = EW // CH
CHP = 128
NPRE = EW // CHP
NPAD = 10240
NSUB = NPAD // NS
TRASH = N
BE = 2048
BN = 1000


def _sc_mesh():
    return plsc.VectorSubcoreMesh(core_axis_name="c", subcore_axis_name="s",
                                  num_cores=NC, num_subcores=NS)


def _pre_body(posx_h, posy_h, posz_h, an_h, j_h, i_h, d2_h, idx_h, te_h,
              jv, iv, xj, yj, zj, xi, yi, zi, tv, d2v, idxv, tev, sem):
    w = lax.axis_index("s") * NC + lax.axis_index("c")
    base = w * EW
    pltpu.sync_copy(j_h.at[pl.ds(base, EW)], jv)
    pltpu.sync_copy(i_h.at[pl.ds(base, EW)], iv)

    def chunk(c, carry):
        s = c * CHP
        jc = jv.at[pl.ds(s, CHP)]
        ic = iv.at[pl.ds(s, CHP)]
        ds = pl.ds(s, CHP)
        descs = [
            pltpu.async_copy(posx_h.at[jc], xj.at[ds], sem),
            pltpu.async_copy(posy_h.at[jc], yj.at[ds], sem),
            pltpu.async_copy(posz_h.at[jc], zj.at[ds], sem),
            pltpu.async_copy(posx_h.at[ic], xi.at[ds], sem),
            pltpu.async_copy(posy_h.at[ic], yi.at[ds], sem),
            pltpu.async_copy(posz_h.at[ic], zi.at[ds], sem),
            pltpu.async_copy(an_h.at[ic], tv.at[ds], sem),
        ]
        for d in descs:
            d.wait()
        return carry

    lax.fori_loop(0, NPRE, chunk, 0)

    def q_body(q, carry):
        s = q * 16
        sl = pl.ds(s, 16)
        dx = xj[sl] - xi[sl]
        dy = yj[sl] - yi[sl]
        dz = zj[sl] - zi[sl]
        d2v[sl] = dx * dx + dy * dy + dz * dz
        t16 = tv[sl]
        idxv[sl] = t16 * N + jv[sl]
        tev[sl] = t16
        return carry

    lax.fori_loop(0, EW // 16, q_body, 0)
    pltpu.sync_copy(d2v, d2_h.at[pl.ds(base, EW)])
    pltpu.sync_copy(idxv, idx_h.at[pl.ds(base, EW)])
    pltpu.sync_copy(tev, te_h.at[pl.ds(base, EW)])


def _pre(posx, posy, posz, an, jp, ip):
    k = pl.kernel(
        _pre_body,
        out_type=[jax.ShapeDtypeStruct((EPAD,), jnp.float32),
                  jax.ShapeDtypeStruct((EPAD,), jnp.int32),
                  jax.ShapeDtypeStruct((EPAD,), jnp.int32)],
        mesh=_sc_mesh(),
        scratch_types=[pltpu.VMEM((EW,), jnp.int32),
                       pltpu.VMEM((EW,), jnp.int32),
                       pltpu.VMEM((EW,), jnp.float32),
                       pltpu.VMEM((EW,), jnp.float32),
                       pltpu.VMEM((EW,), jnp.float32),
                       pltpu.VMEM((EW,), jnp.float32),
                       pltpu.VMEM((EW,), jnp.float32),
                       pltpu.VMEM((EW,), jnp.float32),
                       pltpu.VMEM((EW,), jnp.int32),
                       pltpu.VMEM((EW,), jnp.float32),
                       pltpu.VMEM((EW,), jnp.int32),
                       pltpu.VMEM((EW,), jnp.int32),
                       pltpu.SemaphoreType.DMA],
    )
    return k(posx, posy, posz, an, jp, ip)


def _gs_body(tab_h, filt_h, gidx_h, sidx_h, zeros_h, out_h,
             gidx_v, sidx_v, pa, pb, fa, fb, acc,
             sem_pa, sem_pb, sem_fa, sem_fb):
    cid = lax.axis_index("c")
    sid = lax.axis_index("s")
    w = sid * NC + cid
    base = w * NCHUNK
    r0 = sid * NSUB
    pltpu.sync_copy(zeros_h.at[pl.ds(r0, NSUB)], acc.at[pl.ds(r0, NSUB)])
    pltpu.sync_copy(gidx_h.at[pl.ds(w * EW, EW)], gidx_v)
    pltpu.sync_copy(sidx_h.at[pl.ds(base, NCHUNK)], sidx_v)
    plsc.subcore_barrier()

    def fire(c, pbuf, fbuf, psem, fsem):
        pltpu.async_copy(tab_h.at[gidx_v.at[pl.ds(c * CH, CH)]], pbuf, psem)
        pltpu.async_copy(filt_h.at[pl.ds((base + c) * CH, CH)], fbuf, fsem)

    def drain(c, pbuf, fbuf, psem, fsem):
        pltpu.make_async_copy(tab_h.at[gidx_v.at[pl.ds(0, CH)]], pbuf,
                              psem).wait()
        pltpu.make_async_copy(filt_h.at[pl.ds(base * CH, CH)], fbuf,
                              fsem).wait()

    def mul_scatter(c, pbuf, fbuf):
        def row(r, carry):
            for h in range(H // 16):
                sl = pl.ds(h * 16, 16)
                pbuf[r, sl] = pbuf[r, sl] * fbuf[r, sl]
            return carry

        lax.fori_loop(0, CH, row, 0)
        pltpu.sync_copy(pbuf, acc.at[sidx_v.at[c]], add=True)

    fire(0, pa, fa, sem_pa, sem_fa)

    def k_body(k, carry):
        c = 2 * k
        fire(c + 1, pb, fb, sem_pb, sem_fb)
        drain(c, pa, fa, sem_pa, sem_fa)
        mul_scatter(c, pa, fa)

        @pl.when(c + 2 < NCHUNK)
        def _():
            fire(c + 2, pa, fa, sem_pa, sem_fa)

        drain(c + 1, pb, fb, sem_pb, sem_fb)
        mul_scatter(c + 1, pb, fb)
        return carry

    lax.fori_loop(0, NCHUNK // 2, k_body, 0)
    plsc.subcore_barrier()
    pltpu.sync_copy(acc.at[pl.ds(r0, NSUB)], out_h.at[cid, pl.ds(r0, NSUB)])


def _gs(tab, filt, gidx2d, sidx2d, zerosN):
    buf = pltpu.VMEM((CH, H), jnp.float32)
    k = pl.kernel(
        _gs_body,
        out_type=jax.ShapeDtypeStruct((NC, NPAD, H), jnp.float32),
        mesh=_sc_mesh(),
        scratch_types=[pltpu.VMEM((EW,), jnp.int32),
                       pltpu.VMEM((NCHUNK, CH), jnp.int32),
                       buf, buf, buf, buf,
                       pltpu.VMEM_SHARED((NPAD, H), jnp.float32),
                       pltpu.SemaphoreType.DMA, pltpu.SemaphoreType.DMA,
                       pltpu.SemaphoreType.DMA, pltpu.SemaphoreType.DMA],
    )
    return k(tab, filt, gidx2d, sidx2d, zerosN)


def _x0_kernel(an_ref, emb_ref, out_ref):
    a = an_ref[0, 0, :]
    m = a[:, None] == 0
    x0 = jnp.where(m, emb_ref[0:1, :], emb_ref[1:2, :])
    out_ref[0, :, :] = x0
    out_ref[1, :, :] = jnp.zeros_like(x0)


def _x0(an3, embed):
    nb = N // BN
    return pl.pallas_call(
        _x0_kernel,
        grid=(nb,),
        in_specs=[pl.BlockSpec((1, 1, BN), lambda b: (b, 0, 0)),
                  pl.BlockSpec((T, H), lambda b: (0, 0))],
        out_specs=pl.BlockSpec((2, BN, H), lambda b: (0, b, 0)),
        out_shape=jax.ShapeDtypeStruct((2, NPAD, H), jnp.float32),
    )(an3, embed)


def _phi_kernel(xp_ref, w10, b10, w20, b20, w11, b11, w21, b21, out_ref):
    x = xp_ref[0, :, :] + xp_ref[1, :, :]
    h0 = jax.nn.silu(jnp.dot(x, w10[...], preferred_element_type=jnp.float32)
                     + b10[...])
    p0 = jnp.dot(h0, w20[...], preferred_element_type=jnp.float32) + b20[...]
    h1 = jax.nn.silu(jnp.dot(x, w11[...], preferred_element_type=jnp.float32)
                     + b11[...])
    p1 = jnp.dot(h1, w21[...], preferred_element_type=jnp.float32) + b21[...]
    out_ref[0, :, :] = p0
    out_ref[1, :, :] = p1


def _phi(xp, w10, b10, w20, b20, w11, b11, w21, b21):
    nb = N // BN
    wspec = pl.BlockSpec((H, H), lambda b: (0, 0))
    bspec = pl.BlockSpec((1, H), lambda b: (0, 0))
    return pl.pallas_call(
        _phi_kernel,
        grid=(nb,),
        in_specs=[pl.BlockSpec((2, BN, H), lambda b: (0, b, 0)),
                  wspec, bspec, wspec, bspec, wspec, bspec, wspec, bspec],
        out_specs=pl.BlockSpec((2, BN, H), lambda b: (0, b, 0)),
        out_shape=jax.ShapeDtypeStruct((2, N, H), jnp.float32),
    )(xp, w10, b10, w20, b20, w11, b11, w21, b21)


def _filt_kernel(d2_ref, te_ref, wf0, bf0, wf1, bf1, out_ref):
    d2 = d2_ref[0, 0, :]
    d = jnp.sqrt(d2)
    d = jnp.where(d <= 1e-6, 1e-6, d)
    u = d * (1.0 / RC)
    u2 = u * u
    u4 = u2 * u2
    u5 = u4 * u
    env = 1.0 - 21.0 * u5 + 35.0 * u5 * u - 15.0 * u5 * u2
    env = jnp.where(u < 1.0, env, 0.0)
    offs = lax.broadcasted_iota(jnp.int32, (1, NRBF), 1).astype(jnp.float32) * (
        1.0 / (NRBF - 1))
    delta = 1.0 / (NRBF - 1)
    coeff = -0.5 / (delta * delta)
    diff = u[:, None] - offs
    rbf = jnp.exp(coeff * (diff * diff)) * env[:, None]
    f0 = jnp.dot(rbf, wf0[...], preferred_element_type=jnp.float32) + bf0[...]
    f1 = jnp.dot(rbf, wf1[...], preferred_element_type=jnp.float32) + bf1[...]
    t = te_ref[0, 0, :]
    out_ref[...] = jnp.where(t[:, None] == 0, f0, f1)


def _filt(d2r, ter, wf0, bf0, wf1, bf1):
    nbe = EPAD // BE
    wspec = pl.BlockSpec((NRBF, H), lambda b: (0, 0))
    bspec = pl.BlockSpec((1, H), lambda b: (0, 0))
    return pl.pallas_call(
        _filt_kernel,
        grid=(nbe,),
        in_specs=[pl.BlockSpec((1, 1, BE), lambda b: (b, 0, 0)),
                  pl.BlockSpec((1, 1, BE), lambda b: (b, 0, 0)),
                  wspec, bspec, wspec, bspec],
        out_specs=pl.BlockSpec((BE, H), lambda b: (b, 0)),
        out_shape=jax.ShapeDtypeStruct((EPAD, H), jnp.float32),
    )(d2r, ter, wf0, bf0, wf1, bf1)


def _out_kernel(xp_ref, w1_ref, b1_ref, w2_ref, b2_ref, batch_ref, out_ref):
    b = pl.program_id(0)
    x = xp_ref[0, :, :] + xp_ref[1, :, :]
    h = jax.nn.silu(jnp.dot(x, w1_ref[...], preferred_element_type=jnp.float32)
                    + b1_ref[...]) * (1.0 / 0.6)
    pa = jnp.dot(h, w2_ref[...], preferred_element_type=jnp.float32) + b2_ref[...]
    bt = batch_ref[0, 0, :]
    oh = (bt[:, None] == lax.broadcasted_iota(jnp.int32, (1, NG), 1)
          ).astype(jnp.float32)
    partial = lax.dot_general(pa, oh, (((0,), (0,)), ((), ())))

    @pl.when(b == 0)
    def _():
        out_ref[...] = partial

    @pl.when(b != 0)
    def _():
        out_ref[...] = out_ref[...] + partial


def _out_stage(xp, Wo1, bo1, Wo2, bo2, batch3):
    nb = N // BN
    out = pl.pallas_call(
        _out_kernel,
        grid=(nb,),
        in_specs=[
            pl.BlockSpec((2, BN, H), lambda b: (0, b, 0)),
            pl.BlockSpec((H, H // 2), lambda b: (0, 0)),
            pl.BlockSpec((1, H // 2), lambda b: (0, 0)),
            pl.BlockSpec((H // 2, 1), lambda b: (0, 0)),
            pl.BlockSpec((1, 1), lambda b: (0, 0)),
            pl.BlockSpec((1, 1, BN), lambda b: (b, 0, 0)),
        ],
        out_specs=pl.BlockSpec((1, NG), lambda b: (0, 0)),
        out_shape=jax.ShapeDtypeStruct((1, NG), jnp.float32),
    )(xp, Wo1, bo1.reshape(1, -1), Wo2, bo2.reshape(1, 1), batch3)
    return out.reshape(NG)


def kernel(pos, embed, Wm1, bm1, Wm2, bm2, Wf, bf, Wo1, bo1, Wo2, bo2,
           atomic_number, edge_index, batch):
    f32 = jnp.float32
    pos = pos.astype(f32)
    posx = pos[:, 0]
    posy = pos[:, 1]
    posz = pos[:, 2]
    an = atomic_number.astype(jnp.int32)
    j = edge_index[0].astype(jnp.int32)
    i = edge_index[1].astype(jnp.int32)
    pad = EPAD - E
    jp = jnp.concatenate([j, jnp.zeros((pad,), jnp.int32)])
    ip_g = jnp.concatenate([i, jnp.zeros((pad,), jnp.int32)])
    ip_s = jnp.concatenate([i, jnp.full((pad,), TRASH, jnp.int32)])

    d2, idx_phi, te = _pre(posx, posy, posz, an, jp, ip_g)

    an3 = an.reshape(N // BN, 1, BN)
    xp = _x0(an3, embed.astype(f32))

    d2r = d2.reshape(EPAD // BE, 1, BE)
    ter = te.reshape(EPAD // BE, 1, BE)
    iscat = ip_s.reshape(EPAD // CH, CH)
    zerosN = jnp.zeros((NPAD, H), f32)
    batch3 = batch.astype(jnp.int32).reshape(N // BN, 1, BN)

    filts = [_filt(d2r, ter,
                   Wf[l, 0, :, :H], bf[l, 0, :H].reshape(1, H),
                   Wf[l, 1, :, :H], bf[l, 1, :H].reshape(1, H))
             for l in range(L)]

    for l in range(L):
        phitab = _phi(xp,
                      Wm1[l, 0], bm1[l, 0].reshape(1, H),
                      Wm2[l, 0, :, :H], bm2[l, 0, :H].reshape(1, H),
                      Wm1[l, 1], bm1[l, 1].reshape(1, H),
                      Wm2[l, 1, :, :H], bm2[l, 1, :H].reshape(1, H))
        xp = _gs(phitab.reshape(2 * N, H), filts[l], idx_phi, iscat, zerosN)

    return _out_stage(xp, Wo1, bo1, Wo2, bo2, batch3)
